# baseline (speedup 1.0000x reference)
import jax
import jax.numpy as jnp
from jax import lax
from jax.experimental import pallas as pl
from jax.experimental.pallas import tpu as pltpu
from jax.experimental.pallas import tpu_sc as plsc

V = 50000
T = 100000
P = 6

NC = 2
NS = 16
NW = NC * NS

C = 800
K = T // C
GIDX = 96
NR = 3 * C // GIDX
LANES = 16


def _body(vtx_hbm, bary_hbm, tri_hbm, out_hbm, idx_v, widx_v, vals_v, bary_v, out_v, sem):
    w = lax.axis_index("s") * NC + lax.axis_index("c")

    def do_chunk(k):
        base = k * C
        pltpu.sync_copy(tri_hbm.at[k], idx_v)
        pltpu.sync_copy(bary_hbm.at[pl.ds(base * 3 * P, C * 3 * P)], bary_v)

        def expand(r, carry):
            for cc in range(GIDX // LANES):
                n = r * GIDX + cc * LANES
                v3 = idx_v[r, pl.ds(cc * LANES, LANES)] * 3
                pos = (lax.iota(jnp.int32, LANES) + n) * 3
                plsc.store_scatter(widx_v, [pos], v3)
                plsc.store_scatter(widx_v, [pos + 1], v3 + 1)
                plsc.store_scatter(widx_v, [pos + 2], v3 + 2)
            return carry

        lax.fori_loop(0, NR, expand, 0)
        pltpu.sync_copy(vtx_hbm.at[widx_v], vals_v)

        def group(g, carry):
            tvec = lax.iota(jnp.int32, LANES) + g * LANES
            tb = tvec * (3 * P)
            tr = tvec * 9
            tc = [
                [plsc.load_gather(vals_v, [tr + (3 * i + j)]) for j in range(3)]
                for i in range(3)
            ]
            for p in range(P):
                b = [plsc.load_gather(bary_v, [tb + (3 * p + i)]) for i in range(3)]
                inv = 1.0 / (b[0] + b[1] + b[2])
                for j in range(3):
                    acc = b[0] * tc[0][j] + b[1] * tc[1][j] + b[2] * tc[2][j]
                    plsc.store_scatter(out_v, [tb + (3 * p + j)], acc * inv)
            return carry

        lax.fori_loop(0, C // LANES, group, 0)
        pltpu.sync_copy(out_v, out_hbm.at[pl.ds(base * 3 * P, C * 3 * P)])

    for j in range((K + NW - 1) // NW):
        k = j * NW + w

        @pl.when(k < K)
        def _():
            do_chunk(k)


def kernel(vertex_coords, bary_coords, triangles):
    tri_r = triangles.reshape(K, NR, GIDX)
    bary_flat = bary_coords.reshape(T * P * 3)
    vtx_flat = vertex_coords.reshape(V * 3)
    mesh = plsc.VectorSubcoreMesh(core_axis_name="c", subcore_axis_name="s")
    out = pl.kernel(
        _body,
        out_type=jax.ShapeDtypeStruct((T * P * 3,), jnp.float32),
        mesh=mesh,
        compiler_params=pltpu.CompilerParams(
            needs_layout_passes=False, use_tc_tiling_on_sc=False
        ),
        scratch_types=[
            pltpu.VMEM((NR, GIDX), jnp.int32),
            pltpu.VMEM((C * 9,), jnp.int32),
            pltpu.VMEM((C * 9,), jnp.float32),
            pltpu.VMEM((C * P * 3,), jnp.float32),
            pltpu.VMEM((C * P * 3,), jnp.float32),
            pltpu.SemaphoreType.DMA,
        ],
    )(vtx_flat, bary_flat, tri_r)
    return out.reshape(T * P, 3)


# speedup vs baseline: 16.7557x; 16.7557x over previous
import jax
import jax.numpy as jnp
from jax import lax
from jax.experimental import pallas as pl
from jax.experimental.pallas import tpu as pltpu
from jax.experimental.pallas import tpu_sc as plsc

V = 50000
T = 100000
P = 6

NC = 2
NS = 16
NW = NC * NS

C = 800
K = T // C
LANES = 16
G = C // LANES


def _body(vtx_hbm, bary_hbm, tri_hbm, out_hbm, idx_v, widx_v, vals_v, bary_v, out_v, sem, gsem):
    w = lax.axis_index("s") * NC + lax.axis_index("c")

    def do_chunk(k):
        d_tri = pltpu.async_copy(tri_hbm.at[k], idx_v, sem)
        d_bary = pltpu.async_copy(bary_hbm.at[k], bary_v, sem)
        d_tri.wait()

        def expand(i):
            def step(g, carry):
                v = idx_v[i, pl.ds(g * LANES, LANES)]
                for j in range(3):
                    widx_v[pl.ds((j * 3 + i) * C + g * LANES, LANES)] = v + j * V
                return carry
            lax.fori_loop(0, G, step, 0)

        for i in range(3):
            expand(i)
        d_g = pltpu.async_copy(vtx_hbm.at[widx_v], vals_v, gsem)
        d_bary.wait()
        d_g.wait()

        def group(g, carry):
            t6 = lax.iota(jnp.int32, LANES) * 6 + g * (6 * LANES)
            tc = [
                [vals_v[pl.ds((j * 3 + i) * C + g * LANES, LANES)] for j in range(3)]
                for i in range(3)
            ]
            for p in range(P):
                b = [
                    bary_v[3 * p + i, pl.ds(g * LANES, LANES)]
                    for i in range(3)
                ]
                inv = 1.0 / (b[0] + b[1] + b[2])
                for j in range(3):
                    acc = b[0] * tc[0][j] + b[1] * tc[1][j] + b[2] * tc[2][j]
                    plsc.store_scatter(out_v, [t6 + (j * 6 * C + p)], acc * inv)
            return carry

        lax.fori_loop(0, G, group, 0)
        for j in range(3):
            pltpu.sync_copy(
                out_v.at[pl.ds(j * 6 * C, 6 * C)],
                out_hbm.at[k, j],
            )

    for j in range((K + NW - 1) // NW):
        k = j * NW + w

        @pl.when(k < K)
        def _():
            do_chunk(k)


def kernel(vertex_coords, bary_coords, triangles):
    vxt_flat = vertex_coords.T.reshape(3 * V)
    bary_t = bary_coords.transpose(1, 2, 0).reshape(3 * P, K, C)
    bary_t = bary_t.transpose(1, 0, 2)
    tri_t = triangles.T.reshape(3, K, C).transpose(1, 0, 2)
    mesh = plsc.VectorSubcoreMesh(core_axis_name="c", subcore_axis_name="s")
    out = pl.kernel(
        _body,
        out_type=jax.ShapeDtypeStruct((K, 3, P * C), jnp.float32),
        mesh=mesh,
        compiler_params=pltpu.CompilerParams(
            needs_layout_passes=False, use_tc_tiling_on_sc=False
        ),
        scratch_types=[
            pltpu.VMEM((3, C), jnp.int32),
            pltpu.VMEM((9 * C,), jnp.int32),
            pltpu.VMEM((9 * C,), jnp.float32),
            pltpu.VMEM((3 * P, C), jnp.float32),
            pltpu.VMEM((3 * P * C,), jnp.float32),
            pltpu.SemaphoreType.DMA,
            pltpu.SemaphoreType.DMA,
        ],
    )(vxt_flat, bary_t, tri_t)
    return out.transpose(1, 0, 2).reshape(3, T * P).T


# speedup vs baseline: 22.3585x; 1.3344x over previous
import jax
import jax.numpy as jnp
from jax import lax
from jax.experimental import pallas as pl
from jax.experimental.pallas import tpu as pltpu
from jax.experimental.pallas import tpu_sc as plsc

V = 50000
T = 100000
P = 6

NC = 2
NS = 16
NW = NC * NS

C = 800
K = T // C
LANES = 16
G = C // LANES


def _body(vtx_hbm, bary_hbm, tri_hbm, out_hbm, idx_v, widx_v, vals_v, bary_v, out_v, sem, gsem):
    w = lax.axis_index("s") * NC + lax.axis_index("c")

    def do_chunk(k):
        base = k * C
        d_tri = pltpu.async_copy(tri_hbm.at[:, pl.ds(base, C)], idx_v, sem)
        d_bary = pltpu.async_copy(bary_hbm.at[:, pl.ds(base, C)], bary_v, sem)
        d_tri.wait()

        def expand(i):
            def step(g, carry):
                v = idx_v[i, pl.ds(g * LANES, LANES)]
                for j in range(3):
                    widx_v[pl.ds((j * 3 + i) * C + g * LANES, LANES)] = v + j * V
                return carry
            lax.fori_loop(0, G, step, 0)

        for i in range(3):
            expand(i)
        d_g = pltpu.async_copy(vtx_hbm.at[widx_v], vals_v, gsem)
        d_bary.wait()
        d_g.wait()

        jconst = [jnp.full((LANES,), j, jnp.int32) for j in range(3)]

        def group(g, carry):
            t6 = lax.iota(jnp.int32, LANES) * 6 + g * (6 * LANES)
            tc = [
                [vals_v[pl.ds((j * 3 + i) * C + g * LANES, LANES)] for j in range(3)]
                for i in range(3)
            ]
            for p in range(P):
                b = [
                    bary_v[3 * p + i, pl.ds(g * LANES, LANES)]
                    for i in range(3)
                ]
                inv = 1.0 / (b[0] + b[1] + b[2])
                for j in range(3):
                    acc = b[0] * tc[0][j] + b[1] * tc[1][j] + b[2] * tc[2][j]
                    plsc.store_scatter(
                        out_v, [jconst[j], t6 + p], acc * inv
                    )
            return carry

        lax.fori_loop(0, G, group, 0)
        pltpu.sync_copy(out_v, out_hbm.at[:, pl.ds(P * base, P * C)])

    for j in range((K + NW - 1) // NW):
        k = j * NW + w

        @pl.when(k < K)
        def _():
            do_chunk(k)


def kernel(vertex_coords, bary_coords, triangles):
    vxt_flat = vertex_coords.T.reshape(3 * V)
    bary_t = bary_coords.transpose(1, 2, 0).reshape(3 * P, T)
    tri_t = triangles.T
    mesh = plsc.VectorSubcoreMesh(core_axis_name="c", subcore_axis_name="s")
    out = pl.kernel(
        _body,
        out_type=jax.ShapeDtypeStruct((3, P * T), jnp.float32),
        mesh=mesh,
        compiler_params=pltpu.CompilerParams(
            needs_layout_passes=False, use_tc_tiling_on_sc=False
        ),
        scratch_types=[
            pltpu.VMEM((3, C), jnp.int32),
            pltpu.VMEM((9 * C,), jnp.int32),
            pltpu.VMEM((9 * C,), jnp.float32),
            pltpu.VMEM((3 * P, C), jnp.float32),
            pltpu.VMEM((3, P * C), jnp.float32),
            pltpu.SemaphoreType.DMA,
            pltpu.SemaphoreType.DMA,
        ],
    )(vxt_flat, bary_t, tri_t)
    return out.T


# speedup vs baseline: 22.7127x; 1.0158x over previous
import jax
import jax.numpy as jnp
from jax import lax
from jax.experimental import pallas as pl
from jax.experimental.pallas import tpu as pltpu
from jax.experimental.pallas import tpu_sc as plsc

V = 50000
T = 100000
P = 6

NC = 2
NS = 16
NW = NC * NS

C = 800
K = T // C
LANES = 16
G = C // LANES
J = (K + NW - 1) // NW


def _body(vtx_hbm, bary_hbm, tri_hbm, out_hbm,
          idx_v, widx_v, vals_v, bary_v, out_v,
          tri_sem0, tri_sem1, bary_sem0, bary_sem1,
          g_sem0, g_sem1, out_sem0, out_sem1):
    tri_sem = (tri_sem0, tri_sem1)
    bary_sem = (bary_sem0, bary_sem1)
    g_sem = (g_sem0, g_sem1)
    out_sem = (out_sem0, out_sem1)
    w = lax.axis_index("s") * NC + lax.axis_index("c")

    def valid(j):
        return (j * NW + w) < K

    def in_descs(j):
        base = (j * NW + w) * C
        s = j % 2
        dt = pltpu.make_async_copy(
            tri_hbm.at[:, pl.ds(base, C)], idx_v.at[s], tri_sem[s])
        db = pltpu.make_async_copy(
            bary_hbm.at[:, pl.ds(base, C)], bary_v.at[s], bary_sem[s])
        return dt, db

    def out_desc(j):
        base = (j * NW + w) * C
        s = j % 2
        return pltpu.make_async_copy(
            out_v.at[s], out_hbm.at[:, pl.ds(P * base, P * C)], out_sem[s])

    @pl.when(valid(0))
    def _():
        dt, db = in_descs(0)
        dt.start()
        db.start()

    for j in range(J):
        s = j % 2

        @pl.when(valid(j))
        def _(j=j, s=s):
            dt, db = in_descs(j)
            dt.wait()

            def expand(i):
                def step(g, carry):
                    v = idx_v[s, i, pl.ds(g * LANES, LANES)]
                    for jj in range(3):
                        widx_v[s, pl.ds((jj * 3 + i) * C + g * LANES, LANES)] = (
                            v + jj * V
                        )
                    return carry
                lax.fori_loop(0, G, step, 0)

            for i in range(3):
                expand(i)
            dg = pltpu.make_async_copy(
                vtx_hbm.at[widx_v.at[s]], vals_v.at[s], g_sem[s])
            dg.start()

            if j + 1 < J:
                @pl.when(valid(j + 1))
                def _():
                    dtn, dbn = in_descs(j + 1)
                    dtn.start()
                    dbn.start()

            db.wait()
            dg.wait()
            if j >= 2:
                out_desc(j - 2).wait()

            jconst = [jnp.full((LANES,), jj, jnp.int32) for jj in range(3)]

            def one_group(g):
                t6 = lax.iota(jnp.int32, LANES) * 6 + g * (6 * LANES)
                tc = [
                    [vals_v[s, pl.ds((jj * 3 + i) * C + g * LANES, LANES)]
                     for jj in range(3)]
                    for i in range(3)
                ]
                for p in range(P):
                    b = [bary_v[s, 3 * p + i, pl.ds(g * LANES, LANES)]
                         for i in range(3)]
                    inv = 1.0 / (b[0] + b[1] + b[2])
                    for jj in range(3):
                        acc = b[0] * tc[0][jj] + b[1] * tc[1][jj] + b[2] * tc[2][jj]
                        plsc.store_scatter(
                            out_v.at[s], [jconst[jj], t6 + p], acc * inv)

            def group2(g2, carry):
                one_group(g2 * 2)
                one_group(g2 * 2 + 1)
                return carry

            lax.fori_loop(0, G // 2, group2, 0)
            out_desc(j).start()

    for j in range(J):
        if j + 2 >= J:
            cond = valid(j)
        else:
            cond = valid(j) & jnp.logical_not(valid(j + 2))

        @pl.when(cond)
        def _(j=j):
            out_desc(j).wait()


def kernel(vertex_coords, bary_coords, triangles):
    vxt_flat = vertex_coords.T.reshape(3 * V)
    bary_t = bary_coords.transpose(1, 2, 0).reshape(3 * P, T)
    tri_t = triangles.T
    mesh = plsc.VectorSubcoreMesh(core_axis_name="c", subcore_axis_name="s")
    out = pl.kernel(
        _body,
        out_type=jax.ShapeDtypeStruct((3, P * T), jnp.float32),
        mesh=mesh,
        compiler_params=pltpu.CompilerParams(
            needs_layout_passes=False, use_tc_tiling_on_sc=False
        ),
        scratch_types=[
            pltpu.VMEM((2, 3, C), jnp.int32),
            pltpu.VMEM((2, 9 * C), jnp.int32),
            pltpu.VMEM((2, 9 * C), jnp.float32),
            pltpu.VMEM((2, 3 * P, C), jnp.float32),
            pltpu.VMEM((2, 3, P * C), jnp.float32),
            pltpu.SemaphoreType.DMA,
            pltpu.SemaphoreType.DMA,
            pltpu.SemaphoreType.DMA,
            pltpu.SemaphoreType.DMA,
            pltpu.SemaphoreType.DMA,
            pltpu.SemaphoreType.DMA,
            pltpu.SemaphoreType.DMA,
            pltpu.SemaphoreType.DMA,
        ],
    )(vxt_flat, bary_t, tri_t)
    return out.T
